# baseline (speedup 1.0000x reference)
import functools

import jax
import jax.numpy as jnp
from jax import lax
from jax.experimental import pallas as pl
from jax.experimental.pallas import tpu as pltpu
from jax.experimental.pallas import tpu_sc as plsc

D = 64
CH = 128


@functools.cache
def _make_gather(B: int):
    info = plsc.get_sparse_core_info()
    NC, NS = info.num_cores, info.num_subcores
    NW = NC * NS
    assert B % (NW * CH) == 0
    b_per_w = B // NW
    nchunks = b_per_w // CH
    mesh = plsc.VectorSubcoreMesh(core_axis_name="c", subcore_axis_name="s")

    @functools.partial(
        pl.kernel,
        mesh=mesh,
        out_type=jax.ShapeDtypeStruct((B, D), jnp.float32),
        compiler_params=pltpu.CompilerParams(use_tc_tiling_on_sc=False),
        scratch_types=[
            pltpu.VMEM((nchunks, CH), jnp.int32),
            pltpu.VMEM((2, CH, D), jnp.float32),
            pltpu.SemaphoreType.DMA,
            pltpu.SemaphoreType.DMA,
        ],
    )
    def gather_kernel(idx_hbm, table_hbm, out_hbm, idx_v, rows_v, g_sem, w_sem):
        wid = lax.axis_index("s") * NC + lax.axis_index("c")
        base = wid * b_per_w
        pltpu.sync_copy(idx_hbm.at[wid], idx_v)

        pltpu.async_copy(table_hbm.at[idx_v.at[0]], rows_v.at[0], g_sem)

        def body(j, carry):
            buf = lax.rem(j, 2)
            nbuf = 1 - buf
            @pl.when(j >= 1)
            def _():
                pltpu.make_async_copy(
                    rows_v.at[nbuf], out_hbm.at[pl.ds(base, CH)], w_sem
                ).wait()
            @pl.when(j + 1 < nchunks)
            def _():
                pltpu.async_copy(
                    table_hbm.at[idx_v.at[j + 1]], rows_v.at[nbuf], g_sem
                )
            pltpu.make_async_copy(
                table_hbm.at[idx_v.at[j]], rows_v.at[buf], g_sem
            ).wait()
            pltpu.async_copy(
                rows_v.at[buf], out_hbm.at[pl.ds(base + j * CH, CH)], w_sem
            )
            return carry

        lax.fori_loop(0, nchunks, body, 0)
        pltpu.make_async_copy(
            rows_v.at[0], out_hbm.at[pl.ds(base, CH)], w_sem
        ).wait()

    return gather_kernel


def kernel(input_, weight):
    Bt, S = input_.shape
    B = Bt * S
    info = plsc.get_sparse_core_info()
    NW = info.num_cores * info.num_subcores
    idx = input_.reshape(NW, B // (NW * CH), CH)
    out = _make_gather(B)(idx, weight)
    return out.reshape(Bt, S, D)


# speedup vs baseline: 1.0125x; 1.0125x over previous
import functools

import jax
import jax.numpy as jnp
from jax import lax
from jax.experimental import pallas as pl
from jax.experimental.pallas import tpu as pltpu
from jax.experimental.pallas import tpu_sc as plsc

D = 64
CH = 128


@functools.cache
def _make_gather(B: int):
    info = plsc.get_sparse_core_info()
    NC, NS = info.num_cores, info.num_subcores
    NW = NC * NS
    assert B % (NW * CH) == 0
    b_per_w = B // NW
    nchunks = b_per_w // CH
    mesh = plsc.VectorSubcoreMesh(core_axis_name="c", subcore_axis_name="s")

    NB = 10
    K = 8
    assert nchunks % NB == 0 and K < NB

    @functools.partial(
        pl.kernel,
        mesh=mesh,
        out_type=jax.ShapeDtypeStruct((B, D), jnp.float32),
        compiler_params=pltpu.CompilerParams(use_tc_tiling_on_sc=False),
        scratch_types=[
            pltpu.VMEM((nchunks, CH), jnp.int32),
            pltpu.VMEM((NB, CH, D), jnp.float32),
            pltpu.SemaphoreType.DMA((NB,)),
            pltpu.SemaphoreType.DMA((NB,)),
        ],
    )
    def gather_kernel(idx_hbm, table_hbm, out_hbm, idx_v, rows_v, g_sems, w_sems):
        wid = lax.axis_index("s") * NC + lax.axis_index("c")
        base = wid * b_per_w
        pltpu.sync_copy(idx_hbm.at[wid], idx_v)

        def start_gather(j, slot):
            pltpu.async_copy(
                table_hbm.at[idx_v.at[j]], rows_v.at[slot], g_sems.at[slot]
            )

        def wait_gather(slot):
            pltpu.make_async_copy(
                table_hbm.at[idx_v.at[0]], rows_v.at[slot], g_sems.at[slot]
            ).wait()

        def wait_write(slot):
            pltpu.make_async_copy(
                rows_v.at[slot], out_hbm.at[pl.ds(base, CH)], w_sems.at[slot]
            ).wait()

        for c in range(K):
            start_gather(c, c)

        def outer(o, carry):
            for b in range(NB):
                j = o * NB + b
                s = (b + K) % NB
                @pl.when(j + K < nchunks)
                def _():
                    @pl.when(j + K >= NB)
                    def _():
                        wait_write(s)
                    start_gather(j + K, s)
                wait_gather(b)
                pltpu.async_copy(
                    rows_v.at[b], out_hbm.at[pl.ds(base + j * CH, CH)], w_sems.at[b]
                )
            return carry

        lax.fori_loop(0, nchunks // NB, outer, 0)
        for b in range(NB):
            wait_write(b)

    return gather_kernel


def kernel(input_, weight):
    Bt, S = input_.shape
    B = Bt * S
    info = plsc.get_sparse_core_info()
    NW = info.num_cores * info.num_subcores
    idx = input_.reshape(NW, B // (NW * CH), CH)
    out = _make_gather(B)(idx, weight)
    return out.reshape(Bt, S, D)
